# baseline (speedup 1.0000x reference)
import functools

import jax
import jax.numpy as jnp
from jax import lax
from jax.experimental import pallas as pl
from jax.experimental.pallas import tpu as pltpu
from jax.experimental.pallas import tpu_sc as plsc

N, C, H, W = 4, 32, 512, 512
HW = H * W
NP = N * HW
NC, NS, L = 2, 16, 16
NW = NC * NS
PTS_PER_TILE = NP // NW
B = 256
CHUNKS = PTS_PER_TILE // B


def _sc_body(xt_hbm, gx_hbm, gy_hbm, out_hbm,
             gxv, gyv, idx, wgt, rows, ob,
             gsem, lsem, osem):
    wid = lax.axis_index("s") * NC + lax.axis_index("c")
    tile_base = wid * PTS_PER_TILE
    n = lax.shift_right_logical(wid, 3)
    base_row = lax.shift_left(n, 18)
    row0 = n * C

    def load_grid(i, b):
        base = tile_base + i * B
        pltpu.make_async_copy(gx_hbm.at[pl.ds(base, B)], gxv[b],
                              lsem[b][0]).start()
        pltpu.make_async_copy(gy_hbm.at[pl.ds(base, B)], gyv[b],
                              lsem[b][1]).start()

    def wait_grid(i, b):
        base = tile_base + i * B
        pltpu.make_async_copy(gx_hbm.at[pl.ds(base, B)], gxv[b],
                              lsem[b][0]).wait()
        pltpu.make_async_copy(gy_hbm.at[pl.ds(base, B)], gyv[b],
                              lsem[b][1]).wait()

    def compute_idx(b):
        @plsc.parallel_loop(0, B // L, 1, unroll=1)
        def j_body(j):
            s = pl.ds(j * L, L)
            gxs = gxv[b][s]
            gys = gyv[b][s]
            ix = (gxs + 1.0) * (W * 0.5) - 0.5
            iy = (gys + 1.0) * (H * 0.5) - 0.5
            ixt = ix.astype(jnp.int32).astype(jnp.float32)
            iyt = iy.astype(jnp.int32).astype(jnp.float32)
            ix0 = jnp.where(ix < ixt, ixt - 1.0, ixt)
            iy0 = jnp.where(iy < iyt, iyt - 1.0, iyt)
            fx1 = ix - ix0
            fy1 = iy - iy0
            fx0 = 1.0 - fx1
            fy0 = 1.0 - fy1
            ix1 = ix0 + 1.0
            iy1 = iy0 + 1.0

            vx0 = (ix0 >= 0.0) & (ix0 <= W - 1.0)
            vx1 = (ix1 >= 0.0) & (ix1 <= W - 1.0)
            vy0 = (iy0 >= 0.0) & (iy0 <= H - 1.0)
            vy1 = (iy1 >= 0.0) & (iy1 <= H - 1.0)

            cx0 = jnp.clip(ix0, 0.0, W - 1.0).astype(jnp.int32)
            cx1 = jnp.clip(ix1, 0.0, W - 1.0).astype(jnp.int32)
            cy0w = jnp.clip(iy0, 0.0, H - 1.0).astype(jnp.int32) * W + base_row
            cy1w = jnp.clip(iy1, 0.0, H - 1.0).astype(jnp.int32) * W + base_row

            idx[b][0][s] = cy0w + cx0
            idx[b][1][s] = cy0w + cx1
            idx[b][2][s] = cy1w + cx0
            idx[b][3][s] = cy1w + cx1
            wgt[b][0][s] = jnp.where(vy0 & vx0, fy0 * fx0, 0.0)
            wgt[b][1][s] = jnp.where(vy0 & vx1, fy0 * fx1, 0.0)
            wgt[b][2][s] = jnp.where(vy1 & vx0, fy1 * fx0, 0.0)
            wgt[b][3][s] = jnp.where(vy1 & vx1, fy1 * fx1, 0.0)

    def fire_gathers(b):
        for k in range(4):
            pltpu.make_async_copy(xt_hbm.at[idx[b][k]], rows[b][k],
                                  gsem[b][k]).start()

    def wait_gathers(b):
        for k in range(4):
            pltpu.make_async_copy(xt_hbm.at[idx[b][k]], rows[b][k],
                                  gsem[b][k]).wait()

    def combine(b):
        pidx0 = lax.iota(jnp.int32, L)
        CP = C // 2

        @plsc.parallel_loop(0, B // L, 1, unroll=1)
        def grp_body(j):
            s = pl.ds(j * L, L)
            pidx = pidx0 + j * L
            w0 = plsc.pack(wgt[b][0][s], wgt[b][0][s],
                           format=plsc.PackFormat.INTERLEAVED)
            w1 = plsc.pack(wgt[b][1][s], wgt[b][1][s],
                           format=plsc.PackFormat.INTERLEAVED)
            w2 = plsc.pack(wgt[b][2][s], wgt[b][2][s],
                           format=plsc.PackFormat.INTERLEAVED)
            w3 = plsc.pack(wgt[b][3][s], wgt[b][3][s],
                           format=plsc.PackFormat.INTERLEAVED)
            for cp in range(CP):
                cc = (pidx0 + cp) & (CP - 1)
                v0 = plsc.bitcast(plsc.load_gather(rows[b][0], [pidx, cc]),
                                  jnp.bfloat16)
                v1 = plsc.bitcast(plsc.load_gather(rows[b][1], [pidx, cc]),
                                  jnp.bfloat16)
                v2 = plsc.bitcast(plsc.load_gather(rows[b][2], [pidx, cc]),
                                  jnp.bfloat16)
                v3 = plsc.bitcast(plsc.load_gather(rows[b][3], [pidx, cc]),
                                  jnp.bfloat16)
                acc = w0 * v0 + w1 * v1 + w2 * v2 + w3 * v3
                even, odd = plsc.unpack(acc,
                                        format=plsc.PackFormat.INTERLEAVED)
                cc2 = cc * 2
                plsc.store_scatter(ob[b], [cc2, pidx], even)
                plsc.store_scatter(ob[b], [cc2 + 1, pidx], odd)

    def start_out(i, b):
        lb = pl.multiple_of(tile_base - lax.shift_left(n, 18) + i * B, 256)
        for c in range(C):
            pltpu.make_async_copy(ob[b].at[c], out_hbm.at[row0 + c, pl.ds(lb, B)],
                                  osem[b]).start()

    def wait_out(i, b):
        lb = pl.multiple_of(tile_base - lax.shift_left(n, 18) + i * B, 256)
        for c in range(C):
            pltpu.make_async_copy(ob[b].at[c], out_hbm.at[row0 + c, pl.ds(lb, B)],
                                  osem[b]).wait()

    load_grid(0, 0)
    load_grid(1, 1)

    def main_body(it, _):
        for b in (0, 1):
            i = 2 * it + b

            @pl.when(i < CHUNKS)
            def _():
                wait_grid(i, b)
                compute_idx(b)
                fire_gathers(b)

                @pl.when(i + 2 < CHUNKS)
                def _():
                    load_grid(i + 2, b)

            @pl.when((i >= 1) & (i <= CHUNKS))
            def _():
                j = i - 1
                pb = 1 - b
                wait_gathers(pb)

                @pl.when(j >= 2)
                def _():
                    wait_out(j - 2, pb)

                combine(pb)
                start_out(j, pb)
        return ()

    lax.fori_loop(0, CHUNKS // 2 + 1, main_body, ())

    wait_out(CHUNKS - 2, 0)
    wait_out(CHUNKS - 1, 1)


@jax.jit
def _sc_grid_sample(x_t, gx, gy):
    mesh = plsc.VectorSubcoreMesh(core_axis_name="c", subcore_axis_name="s")

    def body(xt_hbm, gx_hbm, gy_hbm, out_hbm, *scratch):
        gxv = scratch[0:2]
        gyv = scratch[2:4]
        idx = (scratch[4:8], scratch[8:12])
        wgt = (scratch[12:16], scratch[16:20])
        rows = (scratch[20:24], scratch[24:28])
        ob = scratch[28:30]
        gsem = (scratch[30:34], scratch[34:38])
        lsem = (scratch[38:40], scratch[40:42])
        osem = scratch[42:44]
        _sc_body(xt_hbm, gx_hbm, gy_hbm, out_hbm,
                 gxv, gyv, idx, wgt, rows, ob, gsem, lsem, osem)

    scratch_types = (
        [pltpu.VMEM((B,), jnp.float32)] * 4
        + [pltpu.VMEM((B,), jnp.int32)] * 8
        + [pltpu.VMEM((B,), jnp.float32)] * 8
        + [pltpu.VMEM((B, C // 2), jnp.float32)] * 8
        + [pltpu.VMEM((C, B), jnp.float32)] * 2
        + [pltpu.SemaphoreType.DMA] * 14
    )
    f = pl.kernel(
        body,
        out_type=jax.ShapeDtypeStruct((N * C, HW), jnp.float32),
        mesh=mesh,
        scratch_types=scratch_types,
        compiler_params=pltpu.CompilerParams(
            use_tc_tiling_on_sc=False, needs_layout_passes=False),
    )
    return f(x_t, gx, gy)


def kernel(x, grid):
    x_tb = jnp.transpose(x, (0, 2, 3, 1)).astype(jnp.bfloat16)
    x_t = lax.bitcast_convert_type(x_tb.reshape(NP, C // 2, 2),
                                   jnp.float32)
    gx = grid[..., 0].reshape(NP)
    gy = grid[..., 1].reshape(NP)
    out = _sc_grid_sample(x_t, gx, gy)
    return out.reshape(N, C, H, W)


# speedup vs baseline: 1.3839x; 1.3839x over previous
import functools

import jax
import jax.numpy as jnp
from jax import lax
from jax.experimental import pallas as pl
from jax.experimental.pallas import tpu as pltpu
from jax.experimental.pallas import tpu_sc as plsc

N, C, H, W = 4, 32, 512, 512
HW = H * W
NP = N * HW
NC, NS, L = 2, 16, 16
NW = NC * NS
PTS_PER_TILE = NP // NW
B = 256
CHUNKS = PTS_PER_TILE // B


def _sc_body(xt_hbm, gx_hbm, gy_hbm, out_hbm,
             gxv, gyv, idx, wgt, rows, ob,
             gsem, lsem, osem):
    wid = lax.axis_index("s") * NC + lax.axis_index("c")
    tile_base = wid * PTS_PER_TILE
    n = lax.shift_right_logical(wid, 3)
    base_row = lax.shift_left(n, 18)
    row0 = n * C

    def load_grid(i, b):
        base = tile_base + i * B
        pltpu.make_async_copy(gx_hbm.at[pl.ds(base, B)], gxv[b],
                              lsem[b][0]).start()
        pltpu.make_async_copy(gy_hbm.at[pl.ds(base, B)], gyv[b],
                              lsem[b][1]).start()

    def wait_grid(i, b):
        base = tile_base + i * B
        pltpu.make_async_copy(gx_hbm.at[pl.ds(base, B)], gxv[b],
                              lsem[b][0]).wait()
        pltpu.make_async_copy(gy_hbm.at[pl.ds(base, B)], gyv[b],
                              lsem[b][1]).wait()

    def compute_idx(b):
        @plsc.parallel_loop(0, B // L, 1, unroll=1)
        def j_body(j):
            s = pl.ds(j * L, L)
            gxs = gxv[b][s]
            gys = gyv[b][s]
            ix = (gxs + 1.0) * (W * 0.5) - 0.5
            iy = (gys + 1.0) * (H * 0.5) - 0.5
            ixt = ix.astype(jnp.int32).astype(jnp.float32)
            iyt = iy.astype(jnp.int32).astype(jnp.float32)
            ix0 = jnp.where(ix < ixt, ixt - 1.0, ixt)
            iy0 = jnp.where(iy < iyt, iyt - 1.0, iyt)
            fx1 = ix - ix0
            fy1 = iy - iy0
            fx0 = 1.0 - fx1
            fy0 = 1.0 - fy1
            ix1 = ix0 + 1.0
            iy1 = iy0 + 1.0

            vx0 = (ix0 >= 0.0) & (ix0 <= W - 1.0)
            vx1 = (ix1 >= 0.0) & (ix1 <= W - 1.0)
            vy0 = (iy0 >= 0.0) & (iy0 <= H - 1.0)
            vy1 = (iy1 >= 0.0) & (iy1 <= H - 1.0)

            cx0 = jnp.clip(ix0, 0.0, W - 1.0).astype(jnp.int32)
            cx1 = jnp.clip(ix1, 0.0, W - 1.0).astype(jnp.int32)
            cy0w = jnp.clip(iy0, 0.0, H - 1.0).astype(jnp.int32) * W + base_row
            cy1w = jnp.clip(iy1, 0.0, H - 1.0).astype(jnp.int32) * W + base_row

            idx[b][0][s] = cy0w + cx0
            idx[b][1][s] = cy0w + cx1
            idx[b][2][s] = cy1w + cx0
            idx[b][3][s] = cy1w + cx1
            wgt[b][0][s] = jnp.where(vy0 & vx0, fy0 * fx0, 0.0)
            wgt[b][1][s] = jnp.where(vy0 & vx1, fy0 * fx1, 0.0)
            wgt[b][2][s] = jnp.where(vy1 & vx0, fy1 * fx0, 0.0)
            wgt[b][3][s] = jnp.where(vy1 & vx1, fy1 * fx1, 0.0)

    def fire_gathers(b):
        for k in range(4):
            pltpu.make_async_copy(xt_hbm.at[idx[b][k]], rows[b][k],
                                  gsem[b][k]).start()

    def wait_gathers(b):
        for k in range(4):
            pltpu.make_async_copy(xt_hbm.at[idx[b][k]], rows[b][k],
                                  gsem[b][k]).wait()

    def combine(b):
        pidx0 = lax.iota(jnp.int32, L)

        @plsc.parallel_loop(0, B // L, 1, unroll=1)
        def grp_body(j):
            s = pl.ds(j * L, L)
            pidx = pidx0 + j * L
            a0 = wgt[b][0][s]
            a1 = wgt[b][1][s]
            a2 = wgt[b][2][s]
            a3 = wgt[b][3][s]
            for c in range(C):
                cc = (pidx0 + c) & (C - 1)
                v0 = plsc.load_gather(rows[b][0], [pidx, cc])
                v1 = plsc.load_gather(rows[b][1], [pidx, cc])
                v2 = plsc.load_gather(rows[b][2], [pidx, cc])
                v3 = plsc.load_gather(rows[b][3], [pidx, cc])
                acc = a0 * v0 + a1 * v1 + a2 * v2 + a3 * v3
                plsc.store_scatter(ob[b], [cc, pidx], acc)

    def start_out(i, b):
        lb = pl.multiple_of(tile_base - lax.shift_left(n, 18) + i * B, 256)
        for c in range(C):
            pltpu.make_async_copy(ob[b].at[c], out_hbm.at[row0 + c, pl.ds(lb, B)],
                                  osem[b]).start()

    def wait_out(i, b):
        lb = pl.multiple_of(tile_base - lax.shift_left(n, 18) + i * B, 256)
        for c in range(C):
            pltpu.make_async_copy(ob[b].at[c], out_hbm.at[row0 + c, pl.ds(lb, B)],
                                  osem[b]).wait()

    load_grid(0, 0)
    load_grid(1, 1)

    def main_body(it, _):
        for b in (0, 1):
            i = 2 * it + b

            @pl.when(i < CHUNKS)
            def _():
                wait_grid(i, b)
                compute_idx(b)
                fire_gathers(b)

                @pl.when(i + 2 < CHUNKS)
                def _():
                    load_grid(i + 2, b)

            @pl.when((i >= 1) & (i <= CHUNKS))
            def _():
                j = i - 1
                pb = 1 - b
                wait_gathers(pb)

                @pl.when(j >= 2)
                def _():
                    wait_out(j - 2, pb)

                combine(pb)
                start_out(j, pb)
        return ()

    lax.fori_loop(0, CHUNKS // 2 + 1, main_body, ())

    wait_out(CHUNKS - 2, 0)
    wait_out(CHUNKS - 1, 1)


@jax.jit
def _sc_grid_sample(x_t, gx, gy):
    mesh = plsc.VectorSubcoreMesh(core_axis_name="c", subcore_axis_name="s")

    def body(xt_hbm, gx_hbm, gy_hbm, out_hbm, *scratch):
        gxv = scratch[0:2]
        gyv = scratch[2:4]
        idx = (scratch[4:8], scratch[8:12])
        wgt = (scratch[12:16], scratch[16:20])
        rows = (scratch[20:24], scratch[24:28])
        ob = scratch[28:30]
        gsem = (scratch[30:34], scratch[34:38])
        lsem = (scratch[38:40], scratch[40:42])
        osem = scratch[42:44]
        _sc_body(xt_hbm, gx_hbm, gy_hbm, out_hbm,
                 gxv, gyv, idx, wgt, rows, ob, gsem, lsem, osem)

    scratch_types = (
        [pltpu.VMEM((B,), jnp.float32)] * 4
        + [pltpu.VMEM((B,), jnp.int32)] * 8
        + [pltpu.VMEM((B,), jnp.float32)] * 8
        + [pltpu.VMEM((B, C), jnp.float32)] * 8
        + [pltpu.VMEM((C, B), jnp.float32)] * 2
        + [pltpu.SemaphoreType.DMA] * 14
    )
    f = pl.kernel(
        body,
        out_type=jax.ShapeDtypeStruct((N * C, HW), jnp.float32),
        mesh=mesh,
        scratch_types=scratch_types,
        compiler_params=pltpu.CompilerParams(
            use_tc_tiling_on_sc=False, needs_layout_passes=False),
    )
    return f(x_t, gx, gy)


def kernel(x, grid):
    x_t = jnp.transpose(x, (0, 2, 3, 1)).reshape(NP, C)
    gx = grid[..., 0].reshape(NP)
    gy = grid[..., 1].reshape(NP)
    out = _sc_grid_sample(x_t, gx, gy)
    return out.reshape(N, C, H, W)
